# baseline (speedup 1.0000x reference)
import jax
import jax.numpy as jnp
from jax import lax
from jax.experimental import pallas as pl
from jax.experimental.pallas import tpu as pltpu

B, V, F = 4, 2048, 64
K = 40
NDIM, NPROP, NFILT = 4, 64, 128

RBLK = 256
NBLK = V // RBLK


def _grav_kernel(x_ref, Wf_ref, bf_ref, Ws_ref, bs_ref, Wo_ref, bo_ref,
                 out_ref, feat_ref, featb_ref):
    rb = pl.program_id(1)
    x_full = x_ref[0]
    feat = jnp.dot(x_full, Wf_ref[...],
                   preferred_element_type=jnp.float32) + bf_ref[...][None, :]
    feat_ref[...] = feat
    featb_ref[...] = feat.astype(jnp.bfloat16)
    coordsT = lax.dot_general(Ws_ref[...], x_full, (((0,), (1,)), ((), ())),
                              preferred_element_type=jnp.float32) \
        + bs_ref[...][:, None]
    cnT = jnp.sum(coordsT * coordsT, axis=0, keepdims=True)

    x_blk = x_ref[0, pl.ds(rb * RBLK, RBLK), :]
    c_blk = jnp.dot(x_blk, Ws_ref[...],
                    preferred_element_type=jnp.float32) + bs_ref[...][None, :]
    rn = jnp.sum(c_blk * c_blk, axis=1, keepdims=True)
    cross = lax.dot_general(c_blk, coordsT, (((1,), (0,)), ((), ())),
                            preferred_element_type=jnp.float32)
    D0 = (rn - 2.0 * cross) + cnT

    def extract(s):
        m = jnp.min(s, axis=1, keepdims=True)
        eq = s == m
        s = jnp.where(eq, jnp.inf, s)
        H = jnp.where(eq, 1.0, 0.0).astype(jnp.bfloat16)
        G = jnp.dot(H, featb_ref[...],
                    preferred_element_type=jnp.float32)
        w = jnp.exp(-10.0 * jnp.abs(m))
        return w * G, s

    _, s0 = extract(D0)

    def body(k, carry):
        amax, asum, s = carry
        for _ in range(13):
            wG, s = extract(s)
            amax = jnp.maximum(amax, wG)
            asum = asum + wG
        return amax, asum, s

    init = (jnp.full((RBLK, NPROP), -jnp.inf, jnp.float32),
            jnp.zeros((RBLK, NPROP), jnp.float32), s0)
    nmax, nsum, _ = lax.fori_loop(0, (K - 1) // 13, body, init)

    nmean = nsum * (1.0 / (K - 1))
    acc = (jnp.dot(x_blk, Wo_ref[0:F, :], preferred_element_type=jnp.float32)
           + jnp.dot(nmax, Wo_ref[F:F + NPROP, :],
                     preferred_element_type=jnp.float32)
           + jnp.dot(nmean, Wo_ref[F + NPROP:, :],
                     preferred_element_type=jnp.float32)
           + bo_ref[...][None, :])
    out_ref[0] = jnp.tanh(acc)


@jax.jit
def kernel(x, W_f, b_f, W_s, b_s, W_o, b_o):
    grid = (B, NBLK)
    return pl.pallas_call(
        _grav_kernel,
        grid=grid,
        in_specs=[
            pl.BlockSpec((1, V, F), lambda b, r: (b, 0, 0)),
            pl.BlockSpec((F, NPROP), lambda b, r: (0, 0)),
            pl.BlockSpec((NPROP,), lambda b, r: (0,)),
            pl.BlockSpec((F, NDIM), lambda b, r: (0, 0)),
            pl.BlockSpec((NDIM,), lambda b, r: (0,)),
            pl.BlockSpec((F + 2 * NPROP, NFILT), lambda b, r: (0, 0)),
            pl.BlockSpec((NFILT,), lambda b, r: (0,)),
        ],
        out_specs=pl.BlockSpec((1, RBLK, NFILT), lambda b, r: (b, r, 0)),
        out_shape=jax.ShapeDtypeStruct((B, V, NFILT), jnp.float32),
        scratch_shapes=[
            pltpu.VMEM((V, NPROP), jnp.float32),
            pltpu.VMEM((V, NPROP), jnp.bfloat16),
        ],
    )(x, W_f, b_f, W_s, b_s, W_o, b_o)


# speedup vs baseline: 1.2899x; 1.2899x over previous
import jax
import jax.numpy as jnp
from jax import lax
from jax.experimental import pallas as pl
from jax.experimental.pallas import tpu as pltpu

B, V, F = 4, 2048, 64
K = 40
NDIM, NPROP, NFILT = 4, 64, 128

RBLK = 256
NBLK = V // RBLK


def _grav_kernel(x_ref, Wf_ref, bf_ref, Ws_ref, bs_ref, Wo_ref, bo_ref,
                 out_ref, feat_ref):
    rb = pl.program_id(1)
    x_full = x_ref[0]
    feat_ref[...] = jnp.dot(x_full, Wf_ref[...],
                            preferred_element_type=jnp.float32) \
        + bf_ref[...][None, :]
    coordsT = lax.dot_general(Ws_ref[...], x_full, (((0,), (1,)), ((), ())),
                              preferred_element_type=jnp.float32) \
        + bs_ref[...][:, None]
    cnT = jnp.sum(coordsT * coordsT, axis=0, keepdims=True)

    x_blk = x_ref[0, pl.ds(rb * RBLK, RBLK), :]
    c_blk = jnp.dot(x_blk, Ws_ref[...],
                    preferred_element_type=jnp.float32) + bs_ref[...][None, :]
    rn = jnp.sum(c_blk * c_blk, axis=1, keepdims=True)
    cross = lax.dot_general(c_blk, coordsT, (((1,), (0,)), ((), ())),
                            preferred_element_type=jnp.float32)
    D0 = (rn - 2.0 * cross) + cnT

    def extract(s):
        m = jnp.min(s, axis=1, keepdims=True)
        eq = s == m
        s = jnp.where(eq, jnp.inf, s)
        H = jnp.where(eq, 1.0, 0.0)
        G = jnp.dot(H, feat_ref[...],
                    preferred_element_type=jnp.float32)
        w = jnp.exp(-10.0 * jnp.abs(m))
        return w * G, s

    _, s0 = extract(D0)

    def body(k, carry):
        amax, asum, s = carry
        for _ in range(13):
            wG, s = extract(s)
            amax = jnp.maximum(amax, wG)
            asum = asum + wG
        return amax, asum, s

    init = (jnp.full((RBLK, NPROP), -jnp.inf, jnp.float32),
            jnp.zeros((RBLK, NPROP), jnp.float32), s0)
    nmax, nsum, _ = lax.fori_loop(0, (K - 1) // 13, body, init)

    nmean = nsum * (1.0 / (K - 1))
    acc = (jnp.dot(x_blk, Wo_ref[0:F, :], preferred_element_type=jnp.float32)
           + jnp.dot(nmax, Wo_ref[F:F + NPROP, :],
                     preferred_element_type=jnp.float32)
           + jnp.dot(nmean, Wo_ref[F + NPROP:, :],
                     preferred_element_type=jnp.float32)
           + bo_ref[...][None, :])
    out_ref[0] = jnp.tanh(acc)


@jax.jit
def kernel(x, W_f, b_f, W_s, b_s, W_o, b_o):
    grid = (B, NBLK)
    return pl.pallas_call(
        _grav_kernel,
        grid=grid,
        in_specs=[
            pl.BlockSpec((1, V, F), lambda b, r: (b, 0, 0)),
            pl.BlockSpec((F, NPROP), lambda b, r: (0, 0)),
            pl.BlockSpec((NPROP,), lambda b, r: (0,)),
            pl.BlockSpec((F, NDIM), lambda b, r: (0, 0)),
            pl.BlockSpec((NDIM,), lambda b, r: (0,)),
            pl.BlockSpec((F + 2 * NPROP, NFILT), lambda b, r: (0, 0)),
            pl.BlockSpec((NFILT,), lambda b, r: (0,)),
        ],
        out_specs=pl.BlockSpec((1, RBLK, NFILT), lambda b, r: (b, r, 0)),
        out_shape=jax.ShapeDtypeStruct((B, V, NFILT), jnp.float32),
        scratch_shapes=[
            pltpu.VMEM((V, NPROP), jnp.float32),
        ],
    )(x, W_f, b_f, W_s, b_s, W_o, b_o)
